# baseline (speedup 1.0000x reference)
import functools

import jax
import jax.numpy as jnp
import numpy as np
from jax.experimental import pallas as pl
from jax.experimental.pallas import tpu as pltpu

RADIUS = 0.25
D_MIN = 0.0
D_MAX = 0.25
NB = 32
EPS = 1e-12
D_FAR = 1000.0

BI = 8
BJ = 128

_CENTERS = np.linspace(D_MIN, D_MAX, NB).astype(np.float32)
_GAMMA = float(np.float32(1.0) / (_CENTERS[1] - _CENTERS[0]) ** 2)


def _cfconv_kernel(ilo_ref, nib_ref, c_ref, ct_ref, f_ref,
                   w1t_ref, w2t_ref, cen_ref, sel_ref, out_ref):
    jb = pl.program_id(0)
    i0 = ilo_ref[jb]
    nib = nib_ref[jb]
    j0 = jb * BJ

    w1t = w1t_ref[...]
    w2t = w2t_ref[...]
    cenc = cen_ref[...]
    hid = w2t.shape[0]
    gamma = _GAMMA

    xj = ct_ref[0:1, pl.ds(j0, BJ)]
    yj = ct_ref[1:2, pl.ds(j0, BJ)]
    zj = ct_ref[2:3, pl.ds(j0, BJ)]
    sqj = ct_ref[3:4, pl.ds(j0, BJ)]
    bj = ct_ref[4:5, pl.ds(j0, BJ)]
    jg = j0 + jax.lax.broadcasted_iota(jnp.int32, (1, BJ), 1)
    ii = jax.lax.broadcasted_iota(jnp.int32, (BI, 1), 0)

    bf = lambda v: v.astype(jnp.bfloat16).astype(jnp.float32)
    xjb, yjb, zjb = bf(xj), bf(yj), bf(zj)

    def body(t, acc):
        ib = i0 + t * BI
        ci = c_ref[pl.ds(ib, BI), :]
        fi = f_ref[pl.ds(ib, BI), :]
        fit = fi.T
        xi = ci[:, 0:1]
        yi = ci[:, 1:2]
        zi = ci[:, 2:3]
        sqi = ci[:, 3:4]
        bi_ = ci[:, 4:5]

        dx = xi - xj
        dy = yi - yj
        dz = zi - zj
        d2_diff = dx * dx + dy * dy + dz * dz
        d2_dot = (sqi + sqj
                  - 2.0 * (bf(xi) * xjb + bf(yi) * yjb + bf(zi) * zjb))
        mask = (bi_ == bj) & ((ib + ii) != jg) & (d2_dot <= RADIUS * RADIUS)
        dmk = jnp.where(mask, jnp.sqrt(d2_diff + EPS), D_FAR)

        fic = jnp.dot(fit, sel_ref[...],
                      precision=jax.lax.Precision.HIGHEST,
                      preferred_element_type=jnp.float32)
        for u in range(BI):
            drow = dmk[u:u + 1, :]
            rbf = jnp.exp(-gamma * (drow - cenc) ** 2)
            h1 = jnp.maximum(
                jnp.dot(w1t, rbf.astype(jnp.bfloat16),
                        preferred_element_type=jnp.float32), 0.0)
            m = jnp.maximum(
                jnp.dot(w2t, h1.astype(jnp.bfloat16),
                        preferred_element_type=jnp.float32), 0.0)
            acc = acc + m * fic[:, u * BJ:(u + 1) * BJ]
        return acc

    acc = jax.lax.fori_loop(0, nib, body,
                            jnp.zeros((hid, BJ), jnp.float32))
    out_ref[...] = acc.T


@functools.partial(jax.jit, static_argnames=())
def kernel(node_feats, coords, batch_index, W1, W2):
    V, HID = node_feats.shape
    NJ = (V + BJ - 1) // BJ
    VP = NJ * BJ

    starts = jnp.minimum(jnp.arange(NJ, dtype=jnp.int32) * BJ, V - 1)
    ends = jnp.minimum(starts + (BJ - 1), V - 1)
    b_lo = batch_index[starts]
    b_hi = batch_index[ends]
    ilo = jnp.searchsorted(batch_index, b_lo, side="left").astype(jnp.int32)
    ihi = jnp.searchsorted(batch_index, b_hi, side="right").astype(jnp.int32)
    ilo_al = (ilo // BI) * BI
    nib = (ihi - ilo_al + BI - 1) // BI

    sq = jnp.sum(coords * coords, axis=-1)
    c_tab = jnp.zeros((VP, 8), jnp.float32)
    c_tab = c_tab.at[:V, 0:3].set(coords)
    c_tab = c_tab.at[:V, 3].set(sq)
    c_tab = c_tab.at[:V, 4].set(batch_index.astype(jnp.float32))
    c_tab = c_tab.at[V:, 4].set(-1.0)
    ct_tab = c_tab.T

    f_tab = jnp.zeros((VP, HID), node_feats.dtype).at[:V].set(node_feats)

    full = lambda shape: pl.BlockSpec(shape, lambda j: (0, 0))
    out = pl.pallas_call(
        _cfconv_kernel,
        grid=(NJ,),
        in_specs=[
            pl.BlockSpec(memory_space=pltpu.SMEM),
            pl.BlockSpec(memory_space=pltpu.SMEM),
            full((VP, 8)),
            full((8, VP)),
            full((VP, HID)),
            full((HID, NB)),
            full((HID, HID)),
            full((NB, 1)),
            full((BI, BI * BJ)),
        ],
        out_specs=pl.BlockSpec((BJ, HID), lambda j: (j, 0)),
        out_shape=jax.ShapeDtypeStruct((VP, HID), jnp.float32),
    )(ilo_al, nib, c_tab, ct_tab, f_tab,
      W1.T.astype(jnp.bfloat16), W2.T.astype(jnp.bfloat16),
      jnp.asarray(_CENTERS).reshape(NB, 1),
      jnp.kron(jnp.eye(BI, dtype=jnp.float32), jnp.ones((1, BJ), jnp.float32)))
    return out[:V]


# speedup vs baseline: 2.4914x; 2.4914x over previous
import functools

import jax
import jax.numpy as jnp
import numpy as np
from jax.experimental import pallas as pl
from jax.experimental.pallas import tpu as pltpu

RADIUS = 0.25
D_MIN = 0.0
D_MAX = 0.25
NB = 32
EPS = 1e-12
D_FAR = 1000.0

BI = 16
BJ = 128

_CENTERS = np.linspace(D_MIN, D_MAX, NB).astype(np.float32)
_GAMMA = float(np.float32(1.0) / (_CENTERS[1] - _CENTERS[0]) ** 2)


def _cfconv_kernel(ilo_ref, nib_ref, c_ref, ct_ref, f_ref,
                   w1t_ref, w2t_ref, cen_ref, sel_ref, out_ref):
    jb = pl.program_id(0)
    i0 = ilo_ref[jb]
    nib = nib_ref[jb]
    j0 = jb * BJ

    w1t = w1t_ref[...]
    w2t = w2t_ref[...]
    cenc = cen_ref[...]
    hid = w2t.shape[0]
    gamma = _GAMMA

    xj = ct_ref[0:1, pl.ds(j0, BJ)]
    yj = ct_ref[1:2, pl.ds(j0, BJ)]
    zj = ct_ref[2:3, pl.ds(j0, BJ)]
    sqj = ct_ref[3:4, pl.ds(j0, BJ)]
    bj = ct_ref[4:5, pl.ds(j0, BJ)]
    jg = j0 + jax.lax.broadcasted_iota(jnp.int32, (1, BJ), 1)
    ii = jax.lax.broadcasted_iota(jnp.int32, (BI, 1), 0)

    bf = lambda v: v.astype(jnp.bfloat16).astype(jnp.float32)
    xjb, yjb, zjb = bf(xj), bf(yj), bf(zj)

    def body(t, acc):
        ib = i0 + t * BI
        ci = c_ref[pl.ds(ib, BI), :]
        fi = f_ref[pl.ds(ib, BI), :]
        fit = fi.T
        xi = ci[:, 0:1]
        yi = ci[:, 1:2]
        zi = ci[:, 2:3]
        sqi = ci[:, 3:4]
        bi_ = ci[:, 4:5]

        dx = xi - xj
        dy = yi - yj
        dz = zi - zj
        d2_diff = dx * dx + dy * dy + dz * dz
        d2_dot = (sqi + sqj
                  - 2.0 * (bf(xi) * xjb + bf(yi) * yjb + bf(zi) * zjb))
        mask = (bi_ == bj) & ((ib + ii) != jg) & (d2_dot <= RADIUS * RADIUS)
        dmk = jnp.where(mask, jnp.sqrt(d2_diff + EPS), D_FAR)

        for u in range(BI):
            drow = dmk[u:u + 1, :]
            rbf = jnp.exp(-gamma * (drow - cenc) ** 2)
            h1 = jnp.maximum(
                jnp.dot(w1t, rbf.astype(jnp.bfloat16),
                        preferred_element_type=jnp.float32), 0.0)
            m = jnp.maximum(
                jnp.dot(w2t, h1.astype(jnp.bfloat16),
                        preferred_element_type=jnp.float32), 0.0)
            acc = acc + m * fit[:, u:u + 1]
        return acc

    acc = jax.lax.fori_loop(0, nib, body,
                            jnp.zeros((hid, BJ), jnp.float32))
    out_ref[...] = acc.T


@functools.partial(jax.jit, static_argnames=())
def kernel(node_feats, coords, batch_index, W1, W2):
    V, HID = node_feats.shape
    NJ = (V + BJ - 1) // BJ
    VP = NJ * BJ

    starts = jnp.minimum(jnp.arange(NJ, dtype=jnp.int32) * BJ, V - 1)
    ends = jnp.minimum(starts + (BJ - 1), V - 1)
    b_lo = batch_index[starts]
    b_hi = batch_index[ends]
    ilo = jnp.searchsorted(batch_index, b_lo, side="left").astype(jnp.int32)
    ihi = jnp.searchsorted(batch_index, b_hi, side="right").astype(jnp.int32)
    ilo_al = (ilo // BI) * BI
    nib = (ihi - ilo_al + BI - 1) // BI

    sq = jnp.sum(coords * coords, axis=-1)
    c_tab = jnp.zeros((VP, 8), jnp.float32)
    c_tab = c_tab.at[:V, 0:3].set(coords)
    c_tab = c_tab.at[:V, 3].set(sq)
    c_tab = c_tab.at[:V, 4].set(batch_index.astype(jnp.float32))
    c_tab = c_tab.at[V:, 4].set(-1.0)
    ct_tab = c_tab.T

    f_tab = jnp.zeros((VP, HID), node_feats.dtype).at[:V].set(node_feats)

    full = lambda shape: pl.BlockSpec(shape, lambda j: (0, 0))
    out = pl.pallas_call(
        _cfconv_kernel,
        grid=(NJ,),
        in_specs=[
            pl.BlockSpec(memory_space=pltpu.SMEM),
            pl.BlockSpec(memory_space=pltpu.SMEM),
            full((VP, 8)),
            full((8, VP)),
            full((VP, HID)),
            full((HID, NB)),
            full((HID, HID)),
            full((NB, 1)),
            full((BI, BI * BJ)),
        ],
        out_specs=pl.BlockSpec((BJ, HID), lambda j: (j, 0)),
        out_shape=jax.ShapeDtypeStruct((VP, HID), jnp.float32),
    )(ilo_al, nib, c_tab, ct_tab, f_tab,
      W1.T.astype(jnp.bfloat16), W2.T.astype(jnp.bfloat16),
      jnp.asarray(_CENTERS).reshape(NB, 1),
      jnp.kron(jnp.eye(BI, dtype=jnp.float32), jnp.ones((1, BJ), jnp.float32)))
    return out[:V]


# speedup vs baseline: 3.4560x; 1.3872x over previous
import functools

import jax
import jax.numpy as jnp
import numpy as np
from jax.experimental import pallas as pl
from jax.experimental.pallas import tpu as pltpu

RADIUS = 0.25
D_MIN = 0.0
D_MAX = 0.25
NB = 32
EPS = 1e-12
D_FAR = 1000.0

BI = 32
BJ = 128

_CENTERS = np.linspace(D_MIN, D_MAX, NB).astype(np.float32)
_GAMMA = float(np.float32(1.0) / (_CENTERS[1] - _CENTERS[0]) ** 2)


def _cfconv_kernel(ilo_ref, nib_ref, c_ref, ct_ref, f_ref,
                   w1t_ref, w2t_ref, cen_ref, sel_ref, out_ref):
    jb = pl.program_id(0)
    i0 = ilo_ref[jb]
    nib = nib_ref[jb]
    j0 = jb * BJ

    w1t = w1t_ref[...]
    w2t = w2t_ref[...]
    cenc = cen_ref[...]
    hid = w2t.shape[0]
    gamma = _GAMMA

    xj = ct_ref[0:1, pl.ds(j0, BJ)]
    yj = ct_ref[1:2, pl.ds(j0, BJ)]
    zj = ct_ref[2:3, pl.ds(j0, BJ)]
    sqj = ct_ref[3:4, pl.ds(j0, BJ)]
    bj = ct_ref[4:5, pl.ds(j0, BJ)]
    jg = j0 + jax.lax.broadcasted_iota(jnp.int32, (1, BJ), 1)
    ii = jax.lax.broadcasted_iota(jnp.int32, (BI, 1), 0)

    bf = lambda v: v.astype(jnp.bfloat16).astype(jnp.float32)
    xjb, yjb, zjb = bf(xj), bf(yj), bf(zj)

    def body(t, acc):
        ib = i0 + t * BI
        ci = c_ref[pl.ds(ib, BI), :]
        fi = f_ref[pl.ds(ib, BI), :]
        fit = fi.T
        xi = ci[:, 0:1]
        yi = ci[:, 1:2]
        zi = ci[:, 2:3]
        sqi = ci[:, 3:4]
        bi_ = ci[:, 4:5]

        dx = xi - xj
        dy = yi - yj
        dz = zi - zj
        d2_diff = dx * dx + dy * dy + dz * dz
        d2_dot = (sqi + sqj
                  - 2.0 * (bf(xi) * xjb + bf(yi) * yjb + bf(zi) * zjb))
        mask = (bi_ == bj) & ((ib + ii) != jg) & (d2_dot <= RADIUS * RADIUS)
        dmk = jnp.where(mask, jnp.sqrt(d2_diff + EPS), D_FAR)

        for u in range(BI):
            drow = dmk[u:u + 1, :]
            rbf = jnp.exp(-gamma * (drow - cenc) ** 2)
            h1 = jnp.maximum(
                jnp.dot(w1t, rbf.astype(jnp.bfloat16),
                        preferred_element_type=jnp.float32), 0.0)
            m = jnp.maximum(
                jnp.dot(w2t, h1.astype(jnp.bfloat16),
                        preferred_element_type=jnp.float32), 0.0)
            acc = acc + m * fit[:, u:u + 1]
        return acc

    acc = jax.lax.fori_loop(0, nib, body,
                            jnp.zeros((hid, BJ), jnp.float32))
    out_ref[...] = acc.T


@functools.partial(jax.jit, static_argnames=())
def kernel(node_feats, coords, batch_index, W1, W2):
    V, HID = node_feats.shape
    NJ = (V + BJ - 1) // BJ
    VP = NJ * BJ

    starts = jnp.minimum(jnp.arange(NJ, dtype=jnp.int32) * BJ, V - 1)
    ends = jnp.minimum(starts + (BJ - 1), V - 1)
    b_lo = batch_index[starts]
    b_hi = batch_index[ends]
    ilo = jnp.searchsorted(batch_index, b_lo, side="left").astype(jnp.int32)
    ihi = jnp.searchsorted(batch_index, b_hi, side="right").astype(jnp.int32)
    ilo_al = (ilo // BI) * BI
    nib = (ihi - ilo_al + BI - 1) // BI

    sq = jnp.sum(coords * coords, axis=-1)
    c_tab = jnp.zeros((VP, 8), jnp.float32)
    c_tab = c_tab.at[:V, 0:3].set(coords)
    c_tab = c_tab.at[:V, 3].set(sq)
    c_tab = c_tab.at[:V, 4].set(batch_index.astype(jnp.float32))
    c_tab = c_tab.at[V:, 4].set(-1.0)
    ct_tab = c_tab.T

    f_tab = jnp.zeros((VP, HID), node_feats.dtype).at[:V].set(node_feats)

    full = lambda shape: pl.BlockSpec(shape, lambda j: (0, 0))
    out = pl.pallas_call(
        _cfconv_kernel,
        grid=(NJ,),
        in_specs=[
            pl.BlockSpec(memory_space=pltpu.SMEM),
            pl.BlockSpec(memory_space=pltpu.SMEM),
            full((VP, 8)),
            full((8, VP)),
            full((VP, HID)),
            full((HID, NB)),
            full((HID, HID)),
            full((NB, 1)),
            full((BI, BI * BJ)),
        ],
        out_specs=pl.BlockSpec((BJ, HID), lambda j: (j, 0)),
        out_shape=jax.ShapeDtypeStruct((VP, HID), jnp.float32),
    )(ilo_al, nib, c_tab, ct_tab, f_tab,
      W1.T.astype(jnp.bfloat16), W2.T.astype(jnp.bfloat16),
      jnp.asarray(_CENTERS).reshape(NB, 1),
      jnp.kron(jnp.eye(BI, dtype=jnp.float32), jnp.ones((1, BJ), jnp.float32)))
    return out[:V]


# speedup vs baseline: 4.1577x; 1.2030x over previous
import functools

import jax
import jax.numpy as jnp
import numpy as np
from jax.experimental import pallas as pl
from jax.experimental.pallas import tpu as pltpu

RADIUS = 0.25
D_MIN = 0.0
D_MAX = 0.25
NB = 32
EPS = 1e-12
D_FAR = 1000.0

BI = 64
BJ = 128

_CENTERS = np.linspace(D_MIN, D_MAX, NB).astype(np.float32)
_GAMMA = float(np.float32(1.0) / (_CENTERS[1] - _CENTERS[0]) ** 2)


def _cfconv_kernel(ilo_ref, nib_ref, c_ref, ct_ref, f_ref,
                   w1t_ref, w2t_ref, cen_ref, sel_ref, out_ref):
    jb = pl.program_id(0)
    i0 = ilo_ref[jb]
    nib = nib_ref[jb]
    j0 = jb * BJ

    w1t = w1t_ref[...]
    w2t = w2t_ref[...]
    cenc = cen_ref[...]
    hid = w2t.shape[0]
    gamma = _GAMMA

    xj = ct_ref[0:1, pl.ds(j0, BJ)]
    yj = ct_ref[1:2, pl.ds(j0, BJ)]
    zj = ct_ref[2:3, pl.ds(j0, BJ)]
    sqj = ct_ref[3:4, pl.ds(j0, BJ)]
    bj = ct_ref[4:5, pl.ds(j0, BJ)]
    jg = j0 + jax.lax.broadcasted_iota(jnp.int32, (1, BJ), 1)
    ii = jax.lax.broadcasted_iota(jnp.int32, (BI, 1), 0)

    bf = lambda v: v.astype(jnp.bfloat16).astype(jnp.float32)
    xjb, yjb, zjb = bf(xj), bf(yj), bf(zj)

    def body(t, acc):
        ib = i0 + t * BI
        ci = c_ref[pl.ds(ib, BI), :]
        fi = f_ref[pl.ds(ib, BI), :]
        fit = fi.T
        xi = ci[:, 0:1]
        yi = ci[:, 1:2]
        zi = ci[:, 2:3]
        sqi = ci[:, 3:4]
        bi_ = ci[:, 4:5]

        dx = xi - xj
        dy = yi - yj
        dz = zi - zj
        d2_diff = dx * dx + dy * dy + dz * dz
        d2_dot = (sqi + sqj
                  - 2.0 * (bf(xi) * xjb + bf(yi) * yjb + bf(zi) * zjb))
        mask = (bi_ == bj) & ((ib + ii) != jg) & (d2_dot <= RADIUS * RADIUS)
        dmk = jnp.where(mask, jnp.sqrt(d2_diff + EPS), D_FAR)

        for u in range(BI):
            drow = dmk[u:u + 1, :]
            rbf = jnp.exp(-gamma * (drow - cenc) ** 2)
            h1 = jnp.maximum(
                jnp.dot(w1t, rbf.astype(jnp.bfloat16),
                        preferred_element_type=jnp.float32), 0.0)
            m = jnp.maximum(
                jnp.dot(w2t, h1.astype(jnp.bfloat16),
                        preferred_element_type=jnp.float32), 0.0)
            acc = acc + m * fit[:, u:u + 1]
        return acc

    acc = jax.lax.fori_loop(0, nib, body,
                            jnp.zeros((hid, BJ), jnp.float32))
    out_ref[...] = acc.T


@functools.partial(jax.jit, static_argnames=())
def kernel(node_feats, coords, batch_index, W1, W2):
    V, HID = node_feats.shape
    NJ = (V + BJ - 1) // BJ
    VP = NJ * BJ

    starts = jnp.minimum(jnp.arange(NJ, dtype=jnp.int32) * BJ, V - 1)
    ends = jnp.minimum(starts + (BJ - 1), V - 1)
    b_lo = batch_index[starts]
    b_hi = batch_index[ends]
    ilo = jnp.searchsorted(batch_index, b_lo, side="left").astype(jnp.int32)
    ihi = jnp.searchsorted(batch_index, b_hi, side="right").astype(jnp.int32)
    ilo_al = (ilo // BI) * BI
    nib = (ihi - ilo_al + BI - 1) // BI

    sq = jnp.sum(coords * coords, axis=-1)
    c_tab = jnp.zeros((VP, 8), jnp.float32)
    c_tab = c_tab.at[:V, 0:3].set(coords)
    c_tab = c_tab.at[:V, 3].set(sq)
    c_tab = c_tab.at[:V, 4].set(batch_index.astype(jnp.float32))
    c_tab = c_tab.at[V:, 4].set(-1.0)
    ct_tab = c_tab.T

    f_tab = jnp.zeros((VP, HID), node_feats.dtype).at[:V].set(node_feats)

    full = lambda shape: pl.BlockSpec(shape, lambda j: (0, 0))
    out = pl.pallas_call(
        _cfconv_kernel,
        grid=(NJ,),
        in_specs=[
            pl.BlockSpec(memory_space=pltpu.SMEM),
            pl.BlockSpec(memory_space=pltpu.SMEM),
            full((VP, 8)),
            full((8, VP)),
            full((VP, HID)),
            full((HID, NB)),
            full((HID, HID)),
            full((NB, 1)),
            full((BI, BI * BJ)),
        ],
        out_specs=pl.BlockSpec((BJ, HID), lambda j: (j, 0)),
        out_shape=jax.ShapeDtypeStruct((VP, HID), jnp.float32),
    )(ilo_al, nib, c_tab, ct_tab, f_tab,
      W1.T.astype(jnp.bfloat16), W2.T.astype(jnp.bfloat16),
      jnp.asarray(_CENTERS).reshape(NB, 1),
      jnp.kron(jnp.eye(BI, dtype=jnp.float32), jnp.ones((1, BJ), jnp.float32)))
    return out[:V]


# speedup vs baseline: 4.4274x; 1.0649x over previous
import functools

import jax
import jax.numpy as jnp
import numpy as np
from jax.experimental import pallas as pl
from jax.experimental.pallas import tpu as pltpu

RADIUS = 0.25
D_MIN = 0.0
D_MAX = 0.25
NB = 32
EPS = 1e-12
D_FAR = 1000.0

BI = 128
BJ = 128

_CENTERS = np.linspace(D_MIN, D_MAX, NB).astype(np.float32)
_GAMMA = float(np.float32(1.0) / (_CENTERS[1] - _CENTERS[0]) ** 2)


def _cfconv_kernel(ilo_ref, nib_ref, c_ref, ct_ref, f_ref,
                   w1t_ref, w2t_ref, cen_ref, sel_ref, out_ref):
    jb = pl.program_id(0)
    i0 = ilo_ref[jb]
    nib = nib_ref[jb]
    j0 = jb * BJ

    w1t = w1t_ref[...]
    w2t = w2t_ref[...]
    cenc = cen_ref[...]
    hid = w2t.shape[0]
    gamma = _GAMMA

    xj = ct_ref[0:1, pl.ds(j0, BJ)]
    yj = ct_ref[1:2, pl.ds(j0, BJ)]
    zj = ct_ref[2:3, pl.ds(j0, BJ)]
    sqj = ct_ref[3:4, pl.ds(j0, BJ)]
    bj = ct_ref[4:5, pl.ds(j0, BJ)]
    jg = j0 + jax.lax.broadcasted_iota(jnp.int32, (1, BJ), 1)
    ii = jax.lax.broadcasted_iota(jnp.int32, (BI, 1), 0)

    bf = lambda v: v.astype(jnp.bfloat16).astype(jnp.float32)
    xjb, yjb, zjb = bf(xj), bf(yj), bf(zj)

    def body(t, acc):
        ib = i0 + t * BI
        ci = c_ref[pl.ds(ib, BI), :]
        fi = f_ref[pl.ds(ib, BI), :]
        fit = fi.T
        xi = ci[:, 0:1]
        yi = ci[:, 1:2]
        zi = ci[:, 2:3]
        sqi = ci[:, 3:4]
        bi_ = ci[:, 4:5]

        dx = xi - xj
        dy = yi - yj
        dz = zi - zj
        d2_diff = dx * dx + dy * dy + dz * dz
        d2_dot = (sqi + sqj
                  - 2.0 * (bf(xi) * xjb + bf(yi) * yjb + bf(zi) * zjb))
        mask = (bi_ == bj) & ((ib + ii) != jg) & (d2_dot <= RADIUS * RADIUS)
        dmk = jnp.where(mask, jnp.sqrt(d2_diff + EPS), D_FAR)

        for u in range(BI):
            drow = dmk[u:u + 1, :]
            rbf = jnp.exp(-gamma * (drow - cenc) ** 2)
            h1 = jnp.maximum(
                jnp.dot(w1t, rbf.astype(jnp.bfloat16),
                        preferred_element_type=jnp.float32), 0.0)
            m = jnp.maximum(
                jnp.dot(w2t, h1.astype(jnp.bfloat16),
                        preferred_element_type=jnp.float32), 0.0)
            acc = acc + m * fit[:, u:u + 1]
        return acc

    acc = jax.lax.fori_loop(0, nib, body,
                            jnp.zeros((hid, BJ), jnp.float32))
    out_ref[...] = acc.T


@functools.partial(jax.jit, static_argnames=())
def kernel(node_feats, coords, batch_index, W1, W2):
    V, HID = node_feats.shape
    NJ = (V + BJ - 1) // BJ
    VP = NJ * BJ

    starts = jnp.minimum(jnp.arange(NJ, dtype=jnp.int32) * BJ, V - 1)
    ends = jnp.minimum(starts + (BJ - 1), V - 1)
    b_lo = batch_index[starts]
    b_hi = batch_index[ends]
    ilo = jnp.searchsorted(batch_index, b_lo, side="left").astype(jnp.int32)
    ihi = jnp.searchsorted(batch_index, b_hi, side="right").astype(jnp.int32)
    ilo_al = (ilo // BI) * BI
    nib = (ihi - ilo_al + BI - 1) // BI

    sq = jnp.sum(coords * coords, axis=-1)
    c_tab = jnp.zeros((VP, 8), jnp.float32)
    c_tab = c_tab.at[:V, 0:3].set(coords)
    c_tab = c_tab.at[:V, 3].set(sq)
    c_tab = c_tab.at[:V, 4].set(batch_index.astype(jnp.float32))
    c_tab = c_tab.at[V:, 4].set(-1.0)
    ct_tab = c_tab.T

    f_tab = jnp.zeros((VP, HID), node_feats.dtype).at[:V].set(node_feats)

    full = lambda shape: pl.BlockSpec(shape, lambda j: (0, 0))
    out = pl.pallas_call(
        _cfconv_kernel,
        grid=(NJ,),
        in_specs=[
            pl.BlockSpec(memory_space=pltpu.SMEM),
            pl.BlockSpec(memory_space=pltpu.SMEM),
            full((VP, 8)),
            full((8, VP)),
            full((VP, HID)),
            full((HID, NB)),
            full((HID, HID)),
            full((NB, 1)),
            full((BI, BI * BJ)),
        ],
        out_specs=pl.BlockSpec((BJ, HID), lambda j: (j, 0)),
        out_shape=jax.ShapeDtypeStruct((VP, HID), jnp.float32),
    )(ilo_al, nib, c_tab, ct_tab, f_tab,
      W1.T.astype(jnp.bfloat16), W2.T.astype(jnp.bfloat16),
      jnp.asarray(_CENTERS).reshape(NB, 1),
      jnp.kron(jnp.eye(BI, dtype=jnp.float32), jnp.ones((1, BJ), jnp.float32)))
    return out[:V]
